# baseline (speedup 1.0000x reference)
import jax
import jax.numpy as jnp
from jax import lax
from jax.experimental import pallas as pl
from jax.experimental.pallas import tpu as pltpu
from jax.experimental.pallas import tpu_sc as plsc

_BLOCK = 8192
_EMBD = 1024
_NC = 2
_NS = 16
_NW = _NC * _NS
_ROWS = _BLOCK // _NW
_CHUNK = 16
_NBUF = 4
_PREF = 2
_NCHUNK = _ROWS // _CHUNK


def _copy_body(wpe_hbm, out_hbm, *scratch):
    bufs = scratch[:_NBUF]
    ld_sems = scratch[_NBUF:2 * _NBUF]
    st_sems = scratch[2 * _NBUF:]
    wid = lax.axis_index("s") * _NC + lax.axis_index("c")
    base = wid * _ROWS
    loads = [None] * _NBUF
    stores = [None] * _NBUF
    for c in range(min(_PREF, _NCHUNK)):
        loads[c % _NBUF] = pltpu.async_copy(
            wpe_hbm.at[pl.ds(base + c * _CHUNK, _CHUNK), :], bufs[c % _NBUF],
            ld_sems[c % _NBUF])
    for c in range(_NCHUNK):
        b = c % _NBUF
        loads[b].wait()
        stores[b] = pltpu.async_copy(bufs[b],
                                     out_hbm.at[pl.ds(base + c * _CHUNK,
                                                      _CHUNK), :],
                                     st_sems[b])
        j = c + _PREF
        if j < _NCHUNK:
            jb = j % _NBUF
            if stores[jb] is not None:
                stores[jb].wait()
                stores[jb] = None
            loads[jb] = pltpu.async_copy(
                wpe_hbm.at[pl.ds(base + j * _CHUNK, _CHUNK), :], bufs[jb],
                ld_sems[jb])
    for b in range(_NBUF):
        if stores[b] is not None:
            stores[b].wait()


def kernel(wpe):
    mesh = plsc.VectorSubcoreMesh(core_axis_name="c", subcore_axis_name="s")
    out = pl.kernel(
        _copy_body,
        out_type=jax.ShapeDtypeStruct((_BLOCK, _EMBD), jnp.float32),
        mesh=mesh,
        scratch_types=(
            [pltpu.VMEM((_CHUNK, _EMBD), jnp.float32)] * _NBUF
            + [pltpu.SemaphoreType.DMA] * (2 * _NBUF)
        ),
    )(wpe)
    return out.reshape(1, _BLOCK, _EMBD)


# speedup vs baseline: 1.0161x; 1.0161x over previous
import jax
import jax.numpy as jnp
from jax import lax
from jax.experimental import pallas as pl
from jax.experimental.pallas import tpu as pltpu
from jax.experimental.pallas import tpu_sc as plsc

_BLOCK = 8192
_EMBD = 1024
_NC = 2
_NS = 16
_NW = _NC * _NS
_ROWS = _BLOCK // _NW
_CHUNK = 16
_NBUF = 4
_NCHUNK = _ROWS // _CHUNK


def _copy_body(wpe_hbm, out_hbm, *scratch):
    bufs = scratch[:_NBUF]
    ld_sems = scratch[_NBUF:2 * _NBUF]
    st_sems = scratch[2 * _NBUF:]
    wid = lax.axis_index("s") * _NC + lax.axis_index("c")
    base = wid * _ROWS
    loads = [None] * _NBUF
    stores = [None] * _NBUF
    for c in range(min(_NBUF, _NCHUNK)):
        loads[c] = pltpu.async_copy(
            wpe_hbm.at[pl.ds(base + c * _CHUNK, _CHUNK), :], bufs[c],
            ld_sems[c])
    for c in range(_NCHUNK):
        b = c % _NBUF
        loads[b].wait()
        row = base + c * _CHUNK
        stores[b] = pltpu.async_copy(bufs[b],
                                     out_hbm.at[pl.ds(row, _CHUNK), :],
                                     st_sems[b])
        nc = c + _NBUF
        if nc < _NCHUNK:
            stores[b].wait()
            row_n = base + nc * _CHUNK
            loads[b] = pltpu.async_copy(wpe_hbm.at[pl.ds(row_n, _CHUNK), :],
                                        bufs[b], ld_sems[b])
    for b in range(min(_NBUF, _NCHUNK)):
        stores[b].wait()


def kernel(wpe):
    mesh = plsc.VectorSubcoreMesh(core_axis_name="c", subcore_axis_name="s")
    out = pl.kernel(
        _copy_body,
        out_type=jax.ShapeDtypeStruct((_BLOCK, _EMBD), jnp.float32),
        mesh=mesh,
        scratch_types=(
            [pltpu.VMEM((_CHUNK, _EMBD), jnp.float32)] * _NBUF
            + [pltpu.SemaphoreType.DMA] * (2 * _NBUF)
        ),
    )(wpe)
    return out.reshape(1, _BLOCK, _EMBD)
